# baseline (speedup 1.0000x reference)
import functools

import jax
import jax.numpy as jnp
from jax import lax
from jax.experimental import pallas as pl
from jax.experimental.pallas import tpu as pltpu
from jax.experimental.pallas import tpu_sc as plsc

NC = 2
NS = 16
K = 128
DW = 16
DH = 64


@functools.lru_cache(maxsize=None)
def _sc_deg(n_pad: int, nchunk: int):
    rpt = n_pad // NS
    mesh = plsc.VectorSubcoreMesh(core_axis_name="c", subcore_axis_name="s")

    def body(dst_hbm, ones_hbm, zeros_hbm, out_hbm, dst_v, ones_v, acc):
        cid = lax.axis_index("c")
        sid = lax.axis_index("s")
        pltpu.sync_copy(dst_hbm.at[sid], dst_v)
        pltpu.sync_copy(ones_hbm, ones_v)
        pltpu.sync_copy(zeros_hbm, acc.at[pl.ds(sid * rpt, rpt)])
        plsc.subcore_barrier()

        @pl.loop(cid, nchunk, step=NC)
        def _(j):
            pltpu.sync_copy(ones_v, acc.at[dst_v.at[j]], add=True)

        plsc.subcore_barrier()
        pltpu.sync_copy(acc.at[pl.ds(sid * rpt, rpt)],
                        out_hbm.at[cid, pl.ds(sid * rpt, rpt)])

    return pl.kernel(
        body,
        out_type=jax.ShapeDtypeStruct((NC, n_pad, DW), jnp.float32),
        mesh=mesh,
        compiler_params=pltpu.CompilerParams(use_tc_tiling_on_sc=False),
        scratch_types=[
            pltpu.VMEM((nchunk, K), jnp.int32),
            pltpu.VMEM((K, DW), jnp.float32),
            pltpu.VMEM_SHARED((n_pad, DW), jnp.float32),
        ],
    )


@functools.lru_cache(maxsize=None)
def _sc_scatter(n_pad: int, nchunk: int):
    rpt = n_pad // NS
    mesh = plsc.VectorSubcoreMesh(core_axis_name="c", subcore_axis_name="s")

    NB = 4
    assert nchunk % NB == 0 and nchunk > 2 * NB

    def body(s_hbm, src_hbm, dst_hbm, zeros_hbm, out_hbm,
             src_v, dst_v, bufs, gsems, ssems, acc):
        cid = lax.axis_index("c")
        sid = lax.axis_index("s")
        pltpu.sync_copy(src_hbm.at[sid], src_v)
        pltpu.sync_copy(dst_hbm.at[sid], dst_v)
        pltpu.sync_copy(zeros_hbm, acc.at[pl.ds(sid * rpt, rpt)])
        plsc.subcore_barrier()

        table = s_hbm.at[cid]

        def gather(j, b):
            pltpu.async_copy(table.at[src_v.at[j]], bufs[b], gsems[b])

        def gather_wait(j, b):
            pltpu.make_async_copy(
                table.at[src_v.at[j]], bufs[b], gsems[b]).wait()

        def scatter(j, b):
            pltpu.async_copy(bufs[b], acc.at[dst_v.at[j]], ssems[b], add=True)

        def scatter_wait(j, b):
            pltpu.make_async_copy(
                bufs[b], acc.at[dst_v.at[j]], ssems[b]).wait()

        for j in range(2):
            gather(j, j)
        for j in range(2):
            gather_wait(j, j)
            scatter(j, j)
            gather(j + 2, j + 2)

        @pl.loop(2, nchunk - 2, step=NB)
        def _(j0):
            for i in range(NB):
                j = j0 + i
                b = (2 + i) % NB
                gather_wait(j, b)
                scatter(j, b)
                scatter_wait(j - 2, (b + 2) % NB)
                gather(j + 2, (b + 2) % NB)

        for t in range(nchunk - 2, nchunk):
            b = t % NB
            gather_wait(t, b)
            scatter(t, b)
        for t in range(nchunk - 4, nchunk):
            scatter_wait(t, t % NB)

        plsc.subcore_barrier()
        pltpu.sync_copy(acc.at[pl.ds(sid * rpt, rpt)],
                        out_hbm.at[cid, pl.ds(sid * rpt, rpt)])

    return pl.kernel(
        body,
        out_type=jax.ShapeDtypeStruct((NC, n_pad, DH), jnp.float32),
        mesh=mesh,
        compiler_params=pltpu.CompilerParams(use_tc_tiling_on_sc=False),
        scratch_types=[
            pltpu.VMEM((nchunk, K), jnp.int32),
            pltpu.VMEM((nchunk, K), jnp.int32),
            tuple(pltpu.VMEM((K, DH), jnp.float32) for _ in range(NB)),
            tuple(pltpu.SemaphoreType.DMA for _ in range(NB)),
            tuple(pltpu.SemaphoreType.DMA for _ in range(NB)),
            pltpu.VMEM_SHARED((n_pad, DH), jnp.float32),
        ],
    )


_RB = 1000


def _split_spec(n_pad, w):
    return pl.BlockSpec((NC, _RB, w), lambda i: (0, i, 0))


def _row_spec(w):
    return pl.BlockSpec((_RB, w), lambda i: (i, 0))


def _tc_pre(x, w1, deg):
    n, d = x.shape

    def body(x_ref, w_ref, deg_ref, s_ref, dinv_ref):
        dsum = deg_ref[0] + deg_ref[1] + 1.0
        dv = lax.rsqrt(dsum)
        dinv_ref[...] = dv
        res = jnp.dot(
            x_ref[...], w_ref[...], preferred_element_type=jnp.float32
        ) * dv[:, :1]
        s_ref[0] = res[:, :DH]
        s_ref[1] = res[:, DH:]

    return pl.pallas_call(
        body,
        grid=(n // _RB,),
        in_specs=[
            _row_spec(d),
            pl.BlockSpec((d, d), lambda i: (0, 0)),
            _split_spec(None, DW),
        ],
        out_specs=[
            _split_spec(None, DH),
            _row_spec(DW),
        ],
        out_shape=[
            jax.ShapeDtypeStruct((NC, n, DH), jnp.float32),
            jax.ShapeDtypeStruct((n, DW), jnp.float32),
        ],
    )(x, w1, deg)


def _tc_mid(p, s, dinv, b, h_prev, w_next):
    n, d = h_prev.shape

    def body(p_ref, s_ref, dinv_ref, b_ref, hp_ref, w_ref, h_ref, sn_ref):
        dv = dinv_ref[...][:, :1]
        agg = jnp.concatenate(
            [p_ref[0] + s_ref[0], p_ref[1] + s_ref[1]], axis=1
        ) * dv + b_ref[...]
        h = jnp.maximum(agg, 0.0) + hp_ref[...]
        h_ref[...] = h
        res = jnp.dot(
            h, w_ref[...], preferred_element_type=jnp.float32) * dv
        sn_ref[0] = res[:, :DH]
        sn_ref[1] = res[:, DH:]

    return pl.pallas_call(
        body,
        grid=(n // _RB,),
        in_specs=[
            _split_spec(None, DH),
            _split_spec(None, DH),
            _row_spec(DW),
            pl.BlockSpec((1, d), lambda i: (0, 0)),
            _row_spec(d),
            pl.BlockSpec((d, d), lambda i: (0, 0)),
        ],
        out_specs=[
            _row_spec(d),
            _split_spec(None, DH),
        ],
        out_shape=[
            jax.ShapeDtypeStruct((n, d), jnp.float32),
            jax.ShapeDtypeStruct((NC, n, DH), jnp.float32),
        ],
    )(p, s, dinv, b, h_prev, w_next)


def _tc_post(p, s, dinv, b, n, d):
    def body(p_ref, s_ref, dinv_ref, b_ref, o_ref):
        dv = dinv_ref[...][:, :1]
        o_ref[...] = jnp.concatenate(
            [p_ref[0] + s_ref[0], p_ref[1] + s_ref[1]], axis=1
        ) * dv + b_ref[...]

    return pl.pallas_call(
        body,
        grid=(n // _RB,),
        in_specs=[
            _split_spec(None, DH),
            _split_spec(None, DH),
            _row_spec(DW),
            pl.BlockSpec((1, d), lambda i: (0, 0)),
        ],
        out_specs=_row_spec(d),
        out_shape=jax.ShapeDtypeStruct((n, d), jnp.float32),
    )(p, s, dinv, b)


def kernel(x, edge_index, W1, b1, W2, b2, W3, b3):
    n, d = x.shape
    e = edge_index.shape[1]

    n_pad = -(-(n + 1) // (NS * 8)) * (NS * 8)
    rpt = n_pad // NS

    nchunk = -(-(-(-e // (NS * K))) // 4) * 4
    e_pad = NS * nchunk * K

    src = jnp.concatenate(
        [edge_index[0], jnp.zeros((e_pad - e,), jnp.int32)]).reshape(
            NS, nchunk, K)
    dst = jnp.concatenate(
        [edge_index[1], jnp.full((e_pad - e,), n, jnp.int32)]).reshape(
            NS, nchunk, K)

    ones_w = jnp.ones((K, DW), jnp.float32)
    zeros_w = jnp.zeros((rpt, DW), jnp.float32)
    zeros_h = jnp.zeros((rpt, DH), jnp.float32)

    deg = _sc_deg(n_pad, nchunk)(dst, ones_w, zeros_w)
    s1, dinv = _tc_pre(x, W1, deg)

    scat = _sc_scatter(n_pad, nchunk)
    b1r, b2r, b3r = (b.reshape(1, d) for b in (b1, b2, b3))

    p = scat(s1, src, dst, zeros_h)
    h1, s2 = _tc_mid(p, s1, dinv, b1r, x, W2)
    p = scat(s2, src, dst, zeros_h)
    h2, s3 = _tc_mid(p, s2, dinv, b2r, h1, W3)
    p = scat(s3, src, dst, zeros_h)
    return _tc_post(p, s3, dinv, b3r, n, d)


# speedup vs baseline: 1.3296x; 1.3296x over previous
import functools

import jax
import jax.numpy as jnp
from jax import lax
from jax.experimental import pallas as pl
from jax.experimental.pallas import tpu as pltpu
from jax.experimental.pallas import tpu_sc as plsc

NC = 2
NS = 16
K = 128
DW = 16
DH = 64


@functools.lru_cache(maxsize=None)
def _sc_deg(n_pad: int, nchunk: int):
    rpt = n_pad // NS
    mesh = plsc.VectorSubcoreMesh(core_axis_name="c", subcore_axis_name="s")

    def body(dst_hbm, ones_hbm, zeros_hbm, out_hbm, dst_v, ones_v, acc):
        cid = lax.axis_index("c")
        sid = lax.axis_index("s")
        pltpu.sync_copy(dst_hbm.at[sid], dst_v)
        pltpu.sync_copy(ones_hbm, ones_v)
        pltpu.sync_copy(zeros_hbm, acc.at[pl.ds(sid * rpt, rpt)])
        plsc.subcore_barrier()

        @pl.loop(cid, nchunk, step=NC)
        def _(j):
            pltpu.sync_copy(ones_v, acc.at[dst_v.at[j]], add=True)

        plsc.subcore_barrier()
        pltpu.sync_copy(acc.at[pl.ds(sid * rpt, rpt)],
                        out_hbm.at[cid, pl.ds(sid * rpt, rpt)])

    return pl.kernel(
        body,
        out_type=jax.ShapeDtypeStruct((NC, n_pad, DW), jnp.float32),
        mesh=mesh,
        compiler_params=pltpu.CompilerParams(use_tc_tiling_on_sc=False),
        scratch_types=[
            pltpu.VMEM((nchunk, K), jnp.int32),
            pltpu.VMEM((K, DW), jnp.float32),
            pltpu.VMEM_SHARED((n_pad, DW), jnp.float32),
        ],
    )


@functools.lru_cache(maxsize=None)
def _sc_scatter(n_pad: int, nchunk: int):
    rpt = n_pad // NS
    mesh = plsc.VectorSubcoreMesh(core_axis_name="c", subcore_axis_name="s")

    NB = 2
    assert nchunk % NB == 0 and nchunk > 2 * NB

    def body(s_hbm, src_hbm, dst_hbm, zeros_hbm, out_hbm,
             src_v, dst_v, bufs, gsems, acc):
        cid = lax.axis_index("c")
        sid = lax.axis_index("s")
        pltpu.sync_copy(src_hbm.at[sid], src_v)
        pltpu.sync_copy(dst_hbm.at[sid], dst_v)
        pltpu.sync_copy(zeros_hbm, acc.at[pl.ds(sid * rpt, rpt)])
        plsc.subcore_barrier()

        table = s_hbm.at[cid]

        def gather(j, b):
            pltpu.async_copy(table.at[src_v.at[j]], bufs[b], gsems[b])

        def gather_wait(j, b):
            pltpu.make_async_copy(
                table.at[src_v.at[j]], bufs[b], gsems[b]).wait()

        def scatter_sync(j, b):
            pltpu.sync_copy(bufs[b], acc.at[dst_v.at[j]], add=True)

        for b in range(NB):
            gather(b, b)

        @pl.loop(0, nchunk - NB, step=NB)
        def _(j0):
            for b in range(NB):
                j = j0 + b
                gather_wait(j, b)
                scatter_sync(j, b)
                gather(j + NB, b)

        for t in range(nchunk - NB, nchunk):
            gather_wait(t, t % NB)
            scatter_sync(t, t % NB)

        plsc.subcore_barrier()
        pltpu.sync_copy(acc.at[pl.ds(sid * rpt, rpt)],
                        out_hbm.at[cid, pl.ds(sid * rpt, rpt)])

    return pl.kernel(
        body,
        out_type=jax.ShapeDtypeStruct((NC, n_pad, DH), jnp.float32),
        mesh=mesh,
        compiler_params=pltpu.CompilerParams(use_tc_tiling_on_sc=False),
        scratch_types=[
            pltpu.VMEM((nchunk, K), jnp.int32),
            pltpu.VMEM((nchunk, K), jnp.int32),
            tuple(pltpu.VMEM((K, DH), jnp.float32) for _ in range(NB)),
            tuple(pltpu.SemaphoreType.DMA for _ in range(NB)),
            pltpu.VMEM_SHARED((n_pad, DH), jnp.float32),
        ],
    )


_RB = 1000


def _split_spec(n_pad, w):
    return pl.BlockSpec((NC, _RB, w), lambda i: (0, i, 0))


def _row_spec(w):
    return pl.BlockSpec((_RB, w), lambda i: (i, 0))


def _tc_mm(x, w1):
    n, d = x.shape

    def body(x_ref, w_ref, t_ref):
        t_ref[...] = jnp.dot(
            x_ref[...], w_ref[...], preferred_element_type=jnp.float32)

    return pl.pallas_call(
        body,
        grid=(n // _RB,),
        in_specs=[
            _row_spec(d),
            pl.BlockSpec((d, d), lambda i: (0, 0)),
        ],
        out_specs=_row_spec(d),
        out_shape=jax.ShapeDtypeStruct((n, d), jnp.float32),
    )(x, w1)


def _tc_scale(t, deg):
    n, d = t.shape

    def body(t_ref, deg_ref, s_ref, dinv_ref):
        dsum = deg_ref[0] + deg_ref[1] + 1.0
        dv = lax.rsqrt(dsum)
        dinv_ref[...] = dv
        res = t_ref[...] * dv[:, :1]
        s_ref[0] = res[:, :DH]
        s_ref[1] = res[:, DH:]

    return pl.pallas_call(
        body,
        grid=(n // _RB,),
        in_specs=[
            _row_spec(d),
            _split_spec(None, DW),
        ],
        out_specs=[
            _split_spec(None, DH),
            _row_spec(DW),
        ],
        out_shape=[
            jax.ShapeDtypeStruct((NC, n, DH), jnp.float32),
            jax.ShapeDtypeStruct((n, DW), jnp.float32),
        ],
    )(t, deg)


def _tc_mid(p, s, dinv, b, h_prev, w_next):
    n, d = h_prev.shape

    def body(p_ref, s_ref, dinv_ref, b_ref, hp_ref, w_ref, h_ref, sn_ref):
        dv = dinv_ref[...][:, :1]
        agg = jnp.concatenate(
            [p_ref[0] + s_ref[0], p_ref[1] + s_ref[1]], axis=1
        ) * dv + b_ref[...]
        h = jnp.maximum(agg, 0.0) + hp_ref[...]
        h_ref[...] = h
        res = jnp.dot(
            h, w_ref[...], preferred_element_type=jnp.float32) * dv
        sn_ref[0] = res[:, :DH]
        sn_ref[1] = res[:, DH:]

    return pl.pallas_call(
        body,
        grid=(n // _RB,),
        in_specs=[
            _split_spec(None, DH),
            _split_spec(None, DH),
            _row_spec(DW),
            pl.BlockSpec((1, d), lambda i: (0, 0)),
            _row_spec(d),
            pl.BlockSpec((d, d), lambda i: (0, 0)),
        ],
        out_specs=[
            _row_spec(d),
            _split_spec(None, DH),
        ],
        out_shape=[
            jax.ShapeDtypeStruct((n, d), jnp.float32),
            jax.ShapeDtypeStruct((NC, n, DH), jnp.float32),
        ],
    )(p, s, dinv, b, h_prev, w_next)


def _tc_post(p, s, dinv, b, n, d):
    def body(p_ref, s_ref, dinv_ref, b_ref, o_ref):
        dv = dinv_ref[...][:, :1]
        o_ref[...] = jnp.concatenate(
            [p_ref[0] + s_ref[0], p_ref[1] + s_ref[1]], axis=1
        ) * dv + b_ref[...]

    return pl.pallas_call(
        body,
        grid=(n // _RB,),
        in_specs=[
            _split_spec(None, DH),
            _split_spec(None, DH),
            _row_spec(DW),
            pl.BlockSpec((1, d), lambda i: (0, 0)),
        ],
        out_specs=_row_spec(d),
        out_shape=jax.ShapeDtypeStruct((n, d), jnp.float32),
    )(p, s, dinv, b)


def kernel(x, edge_index, W1, b1, W2, b2, W3, b3):
    n, d = x.shape
    e = edge_index.shape[1]

    n_pad = -(-(n + 1) // (NS * 8)) * (NS * 8)
    rpt = n_pad // NS

    nchunk = -(-(-(-e // (NS * K))) // 2) * 2
    e_pad = NS * nchunk * K

    src = jnp.concatenate(
        [edge_index[0], jnp.zeros((e_pad - e,), jnp.int32)]).reshape(
            NS, nchunk, K)
    dst = jnp.concatenate(
        [edge_index[1], jnp.full((e_pad - e,), n, jnp.int32)]).reshape(
            NS, nchunk, K)

    ones_w = jnp.ones((K, DW), jnp.float32)
    zeros_w = jnp.zeros((rpt, DW), jnp.float32)
    zeros_h = jnp.zeros((rpt, DH), jnp.float32)

    t1 = _tc_mm(x, W1)
    deg = _sc_deg(n_pad, nchunk)(dst, ones_w, zeros_w)
    s1, dinv = _tc_scale(t1, deg)

    scat = _sc_scatter(n_pad, nchunk)
    b1r, b2r, b3r = (b.reshape(1, d) for b in (b1, b2, b3))

    p = scat(s1, src, dst, zeros_h)
    h1, s2 = _tc_mid(p, s1, dinv, b1r, x, W2)
    p = scat(s2, src, dst, zeros_h)
    h2, s3 = _tc_mid(p, s2, dinv, b2r, h1, W3)
    p = scat(s3, src, dst, zeros_h)
    return _tc_post(p, s3, dinv, b3r, n, d)


# speedup vs baseline: 1.3416x; 1.0091x over previous
import functools

import jax
import jax.numpy as jnp
from jax import lax
from jax.experimental import pallas as pl
from jax.experimental.pallas import tpu as pltpu
from jax.experimental.pallas import tpu_sc as plsc

NC = 2
NS = 16
K = 128
DW = 16
DH = 64


@functools.lru_cache(maxsize=None)
def _sc_deg(n_pad: int, nchunk: int):
    rpt = n_pad // NS
    mesh = plsc.VectorSubcoreMesh(core_axis_name="c", subcore_axis_name="s")

    def body(dst_hbm, ones_hbm, zeros_hbm, out_hbm, dst_v, ones_v, acc):
        cid = lax.axis_index("c")
        sid = lax.axis_index("s")
        pltpu.sync_copy(dst_hbm.at[sid], dst_v)
        pltpu.sync_copy(ones_hbm, ones_v)
        pltpu.sync_copy(zeros_hbm, acc.at[pl.ds(sid * rpt, rpt)])
        plsc.subcore_barrier()

        @pl.loop(cid, nchunk, step=NC)
        def _(j):
            pltpu.sync_copy(ones_v, acc.at[dst_v.at[j]], add=True)

        plsc.subcore_barrier()
        pltpu.sync_copy(acc.at[pl.ds(sid * rpt, rpt)],
                        out_hbm.at[cid, pl.ds(sid * rpt, rpt)])

    return pl.kernel(
        body,
        out_type=jax.ShapeDtypeStruct((NC, n_pad, DW), jnp.float32),
        mesh=mesh,
        compiler_params=pltpu.CompilerParams(use_tc_tiling_on_sc=False),
        scratch_types=[
            pltpu.VMEM((nchunk, K), jnp.int32),
            pltpu.VMEM((K, DW), jnp.float32),
            pltpu.VMEM_SHARED((n_pad, DW), jnp.float32),
        ],
    )


@functools.lru_cache(maxsize=None)
def _sc_scatter(n_pad: int, nchunk: int):
    rpt = n_pad // NS
    mesh = plsc.VectorSubcoreMesh(core_axis_name="c", subcore_axis_name="s")

    NB = 2
    assert nchunk % NB == 0 and nchunk > 2 * NB

    def body(s_hbm, src_hbm, dst_hbm, out_hbm,
             src_v, dst_v, bufs, gsems, acc):
        cid = lax.axis_index("c")
        sid = lax.axis_index("s")
        table = s_hbm.at[cid]
        pltpu.sync_copy(src_hbm.at[sid], src_v)
        pltpu.sync_copy(dst_hbm.at[sid], dst_v)
        pltpu.sync_copy(table.at[pl.ds(sid * rpt, rpt)],
                        acc.at[pl.ds(sid * rpt, rpt)])
        plsc.subcore_barrier()

        def gather(j, b):
            pltpu.async_copy(table.at[src_v.at[j]], bufs[b], gsems[b])

        def gather_wait(j, b):
            pltpu.make_async_copy(
                table.at[src_v.at[j]], bufs[b], gsems[b]).wait()

        def scatter_sync(j, b):
            pltpu.sync_copy(bufs[b], acc.at[dst_v.at[j]], add=True)

        for b in range(NB):
            gather(b, b)

        @pl.loop(0, nchunk - NB, step=NB)
        def _(j0):
            for b in range(NB):
                j = j0 + b
                gather_wait(j, b)
                scatter_sync(j, b)
                gather(j + NB, b)

        for t in range(nchunk - NB, nchunk):
            gather_wait(t, t % NB)
            scatter_sync(t, t % NB)

        plsc.subcore_barrier()
        pltpu.sync_copy(acc.at[pl.ds(sid * rpt, rpt)],
                        out_hbm.at[cid, pl.ds(sid * rpt, rpt)])

    return pl.kernel(
        body,
        out_type=jax.ShapeDtypeStruct((NC, n_pad, DH), jnp.float32),
        mesh=mesh,
        compiler_params=pltpu.CompilerParams(use_tc_tiling_on_sc=False),
        scratch_types=[
            pltpu.VMEM((nchunk, K), jnp.int32),
            pltpu.VMEM((nchunk, K), jnp.int32),
            tuple(pltpu.VMEM((K, DH), jnp.float32) for _ in range(NB)),
            tuple(pltpu.SemaphoreType.DMA for _ in range(NB)),
            pltpu.VMEM_SHARED((n_pad, DH), jnp.float32),
        ],
    )


_RB = 1000


def _split_spec(n_pad, w):
    return pl.BlockSpec((NC, _RB, w), lambda i: (0, i, 0))


def _row_spec(w):
    return pl.BlockSpec((_RB, w), lambda i: (i, 0))


def _tc_pre(x, w1, deg):
    n, d = x.shape

    def body(x_ref, w_ref, deg_ref, s_ref, dinv_ref):
        dsum = deg_ref[0] + deg_ref[1] + 1.0
        dv = lax.rsqrt(dsum)
        dinv_ref[...] = dv
        res = jnp.dot(
            x_ref[...], w_ref[...], preferred_element_type=jnp.float32
        ) * dv[:, :1]
        s_ref[0] = res[:, :DH]
        s_ref[1] = res[:, DH:]

    return pl.pallas_call(
        body,
        grid=(n // _RB,),
        in_specs=[
            _row_spec(d),
            pl.BlockSpec((d, d), lambda i: (0, 0)),
            _split_spec(None, DW),
        ],
        out_specs=[
            _split_spec(None, DH),
            _row_spec(DW),
        ],
        out_shape=[
            jax.ShapeDtypeStruct((NC, -(-(n + 1) // (NS * 8)) * (NS * 8), DH),
                                 jnp.float32),
            jax.ShapeDtypeStruct((n, DW), jnp.float32),
        ],
    )(x, w1, deg)


def _tc_mid(p, dinv, b, h_prev, w_next):
    n, d = h_prev.shape

    def body(p_ref, dinv_ref, b_ref, hp_ref, w_ref, h_ref, sn_ref):
        dv = dinv_ref[...][:, :1]
        agg = jnp.concatenate(
            [p_ref[0], p_ref[1]], axis=1) * dv + b_ref[...]
        h = jnp.maximum(agg, 0.0) + hp_ref[...]
        h_ref[...] = h
        res = jnp.dot(
            h, w_ref[...], preferred_element_type=jnp.float32) * dv
        sn_ref[0] = res[:, :DH]
        sn_ref[1] = res[:, DH:]

    return pl.pallas_call(
        body,
        grid=(n // _RB,),
        in_specs=[
            _split_spec(None, DH),
            _row_spec(DW),
            pl.BlockSpec((1, d), lambda i: (0, 0)),
            _row_spec(d),
            pl.BlockSpec((d, d), lambda i: (0, 0)),
        ],
        out_specs=[
            _row_spec(d),
            _split_spec(None, DH),
        ],
        out_shape=[
            jax.ShapeDtypeStruct((n, d), jnp.float32),
            jax.ShapeDtypeStruct((NC, -(-(n + 1) // (NS * 8)) * (NS * 8), DH),
                                 jnp.float32),
        ],
    )(p, dinv, b, h_prev, w_next)


def _tc_post(p, dinv, b, n, d):
    def body(p_ref, dinv_ref, b_ref, o_ref):
        dv = dinv_ref[...][:, :1]
        o_ref[...] = jnp.concatenate(
            [p_ref[0], p_ref[1]], axis=1) * dv + b_ref[...]

    return pl.pallas_call(
        body,
        grid=(n // _RB,),
        in_specs=[
            _split_spec(None, DH),
            _row_spec(DW),
            pl.BlockSpec((1, d), lambda i: (0, 0)),
        ],
        out_specs=_row_spec(d),
        out_shape=jax.ShapeDtypeStruct((n, d), jnp.float32),
    )(p, dinv, b)


def kernel(x, edge_index, W1, b1, W2, b2, W3, b3):
    n, d = x.shape
    e = edge_index.shape[1]

    n_pad = -(-(n + 1) // (NS * 8)) * (NS * 8)
    rpt = n_pad // NS

    nchunk = -(-(-(-e // (NS * K))) // 2) * 2
    e_pad = NS * nchunk * K

    src = jnp.concatenate(
        [edge_index[0], jnp.zeros((e_pad - e,), jnp.int32)]).reshape(
            NS, nchunk, K)
    dst = jnp.concatenate(
        [edge_index[1], jnp.full((e_pad - e,), n, jnp.int32)]).reshape(
            NS, nchunk, K)

    ones_w = jnp.ones((K, DW), jnp.float32)
    zeros_w = jnp.zeros((rpt, DW), jnp.float32)

    deg = _sc_deg(n_pad, nchunk)(dst, ones_w, zeros_w)
    s1, dinv = _tc_pre(x, W1, deg)

    scat = _sc_scatter(n_pad, nchunk)
    b1r, b2r, b3r = (b.reshape(1, d) for b in (b1, b2, b3))

    p = scat(s1, src, dst)
    h1, s2 = _tc_mid(p, dinv, b1r, x, W2)
    p = scat(s2, src, dst)
    h2, s3 = _tc_mid(p, dinv, b2r, h1, W3)
    p = scat(s3, src, dst)
    return _tc_post(p, dinv, b3r, n, d)


# speedup vs baseline: 1.3705x; 1.0216x over previous
import functools

import jax
import jax.numpy as jnp
from jax import lax
from jax.experimental import pallas as pl
from jax.experimental.pallas import tpu as pltpu
from jax.experimental.pallas import tpu_sc as plsc

NC = 2
NS = 16
K = 128
DW = 16
DH = 64


@functools.lru_cache(maxsize=None)
def _sc_deg(n_pad: int, nchunk: int):
    rpt = n_pad // NS
    mesh = plsc.VectorSubcoreMesh(core_axis_name="c", subcore_axis_name="s")

    def body(dst_hbm, ones_hbm, zeros_hbm, out_hbm, dst_v, ones_v, acc):
        cid = lax.axis_index("c")
        sid = lax.axis_index("s")
        pltpu.sync_copy(dst_hbm.at[sid], dst_v)
        pltpu.sync_copy(ones_hbm, ones_v)
        pltpu.sync_copy(zeros_hbm, acc.at[pl.ds(sid * rpt, rpt)])
        plsc.subcore_barrier()

        @pl.loop(cid, nchunk, step=NC)
        def _(j):
            pltpu.sync_copy(ones_v, acc.at[dst_v.at[j]], add=True)

        plsc.subcore_barrier()
        pltpu.sync_copy(acc.at[pl.ds(sid * rpt, rpt)],
                        out_hbm.at[cid, pl.ds(sid * rpt, rpt)])

    return pl.kernel(
        body,
        out_type=jax.ShapeDtypeStruct((NC, n_pad, DW), jnp.float32),
        mesh=mesh,
        compiler_params=pltpu.CompilerParams(use_tc_tiling_on_sc=False),
        scratch_types=[
            pltpu.VMEM((nchunk, K), jnp.int32),
            pltpu.VMEM((K, DW), jnp.float32),
            pltpu.VMEM_SHARED((n_pad, DW), jnp.float32),
        ],
    )


@functools.lru_cache(maxsize=None)
def _sc_scatter(n_pad: int, nchunk: int):
    rpt = n_pad // NS
    mesh = plsc.VectorSubcoreMesh(core_axis_name="c", subcore_axis_name="s")

    NB = 2
    assert nchunk % NB == 0 and nchunk > 2 * NB

    def body(s_hbm, src_hbm, dst_hbm, zeros_hbm, out_hbm,
             src_v, dst_v, bufs, gsems, acc):
        cid = lax.axis_index("c")
        sid = lax.axis_index("s")
        table = s_hbm.at[cid]
        pltpu.sync_copy(src_hbm.at[sid], src_v)
        pltpu.sync_copy(dst_hbm.at[sid], dst_v)
        pltpu.sync_copy(zeros_hbm, acc.at[pl.ds(sid * rpt, rpt)])
        plsc.subcore_barrier()

        def gather(j, b):
            pltpu.async_copy(table.at[src_v.at[j]], bufs[b], gsems[b])

        def gather_wait(j, b):
            pltpu.make_async_copy(
                table.at[src_v.at[j]], bufs[b], gsems[b]).wait()

        def scatter_sync(j, b):
            pltpu.sync_copy(bufs[b], acc.at[dst_v.at[j]], add=True)

        for b in range(NB):
            gather(b, b)

        @pl.loop(0, nchunk - NB, step=NB)
        def _(j0):
            for b in range(NB):
                j = j0 + b
                gather_wait(j, b)
                scatter_sync(j, b)
                gather(j + NB, b)

        for t in range(nchunk - NB, nchunk):
            gather_wait(t, t % NB)
            scatter_sync(t, t % NB)

        plsc.subcore_barrier()
        pltpu.sync_copy(acc.at[pl.ds(sid * rpt, rpt)],
                        out_hbm.at[cid, pl.ds(sid * rpt, rpt)])

    return pl.kernel(
        body,
        out_type=jax.ShapeDtypeStruct((NC, n_pad, DH), jnp.float32),
        mesh=mesh,
        compiler_params=pltpu.CompilerParams(use_tc_tiling_on_sc=False),
        scratch_types=[
            pltpu.VMEM((nchunk, K), jnp.int32),
            pltpu.VMEM((nchunk, K), jnp.int32),
            tuple(pltpu.VMEM((K, DH), jnp.float32) for _ in range(NB)),
            tuple(pltpu.SemaphoreType.DMA for _ in range(NB)),
            pltpu.VMEM_SHARED((n_pad, DH), jnp.float32),
        ],
    )


_RB = 1000


def _split_spec(n_pad, w):
    return pl.BlockSpec((NC, _RB, w), lambda i: (0, i, 0))


def _row_spec(w):
    return pl.BlockSpec((_RB, w), lambda i: (i, 0))


def _tc_pre(x, w1, deg):
    n, d = x.shape

    def body(x_ref, w_ref, deg_ref, s_ref, dinv_ref):
        dsum = deg_ref[0] + deg_ref[1] + 1.0
        dv = lax.rsqrt(dsum)
        dinv_ref[...] = dv
        res = jnp.dot(
            x_ref[...], w_ref[...], preferred_element_type=jnp.float32
        ) * dv[:, :1]
        s_ref[0] = res[:, :DH]
        s_ref[1] = res[:, DH:]

    return pl.pallas_call(
        body,
        grid=(n // _RB,),
        in_specs=[
            _row_spec(d),
            pl.BlockSpec((d, d), lambda i: (0, 0)),
            _split_spec(None, DW),
        ],
        out_specs=[
            _split_spec(None, DH),
            _row_spec(DW),
        ],
        out_shape=[
            jax.ShapeDtypeStruct((NC, n, DH), jnp.float32),
            jax.ShapeDtypeStruct((n, DW), jnp.float32),
        ],
    )(x, w1, deg)


def _tc_mid(p, s, dinv, b, h_prev, w_next):
    n, d = h_prev.shape

    def body(p_ref, s_ref, dinv_ref, b_ref, hp_ref, w_ref, h_ref, sn_ref):
        dv = dinv_ref[...][:, :1]
        agg = jnp.concatenate(
            [p_ref[0] + s_ref[0], p_ref[1] + s_ref[1]], axis=1
        ) * dv + b_ref[...]
        h = jnp.maximum(agg, 0.0) + hp_ref[...]
        h_ref[...] = h
        res = jnp.dot(
            h, w_ref[...], preferred_element_type=jnp.float32) * dv
        sn_ref[0] = res[:, :DH]
        sn_ref[1] = res[:, DH:]

    return pl.pallas_call(
        body,
        grid=(n // _RB,),
        in_specs=[
            _split_spec(None, DH),
            _split_spec(None, DH),
            _row_spec(DW),
            pl.BlockSpec((1, d), lambda i: (0, 0)),
            _row_spec(d),
            pl.BlockSpec((d, d), lambda i: (0, 0)),
        ],
        out_specs=[
            _row_spec(d),
            _split_spec(None, DH),
        ],
        out_shape=[
            jax.ShapeDtypeStruct((n, d), jnp.float32),
            jax.ShapeDtypeStruct((NC, n, DH), jnp.float32),
        ],
    )(p, s, dinv, b, h_prev, w_next)


def _tc_post(p, s, dinv, b, n, d):
    def body(p_ref, s_ref, dinv_ref, b_ref, o_ref):
        dv = dinv_ref[...][:, :1]
        o_ref[...] = jnp.concatenate(
            [p_ref[0] + s_ref[0], p_ref[1] + s_ref[1]], axis=1
        ) * dv + b_ref[...]

    return pl.pallas_call(
        body,
        grid=(n // _RB,),
        in_specs=[
            _split_spec(None, DH),
            _split_spec(None, DH),
            _row_spec(DW),
            pl.BlockSpec((1, d), lambda i: (0, 0)),
        ],
        out_specs=_row_spec(d),
        out_shape=jax.ShapeDtypeStruct((n, d), jnp.float32),
    )(p, s, dinv, b)


def kernel(x, edge_index, W1, b1, W2, b2, W3, b3):
    n, d = x.shape
    e = edge_index.shape[1]

    n_pad = -(-(n + 1) // (NS * 8)) * (NS * 8)
    rpt = n_pad // NS

    nchunk = -(-(-(-e // (NS * K))) // 2) * 2
    e_pad = NS * nchunk * K

    src = jnp.concatenate(
        [edge_index[0], jnp.zeros((e_pad - e,), jnp.int32)]).reshape(
            NS, nchunk, K)
    dst = jnp.concatenate(
        [edge_index[1], jnp.full((e_pad - e,), n, jnp.int32)]).reshape(
            NS, nchunk, K)

    ones_w = jnp.ones((K, DW), jnp.float32)
    zeros_w = jnp.zeros((rpt, DW), jnp.float32)
    zeros_h = jnp.zeros((rpt, DH), jnp.float32)

    deg = _sc_deg(n_pad, nchunk)(dst, ones_w, zeros_w)
    s1, dinv = _tc_pre(x, W1, deg)

    scat = _sc_scatter(n_pad, nchunk)
    b1r, b2r, b3r = (b.reshape(1, d) for b in (b1, b2, b3))

    p = scat(s1, src, dst, zeros_h)
    h1, s2 = _tc_mid(p, s1, dinv, b1r, x, W2)
    p = scat(s2, src, dst, zeros_h)
    h2, s3 = _tc_mid(p, s2, dinv, b2r, h1, W3)
    p = scat(s3, src, dst, zeros_h)
    return _tc_post(p, s3, dinv, b3r, n, d)


# speedup vs baseline: 1.3757x; 1.0037x over previous
import functools

import jax
import jax.numpy as jnp
from jax import lax
from jax.experimental import pallas as pl
from jax.experimental.pallas import tpu as pltpu
from jax.experimental.pallas import tpu_sc as plsc

NC = 2
NS = 16
K = 128
DW = 16
DH = 64


@functools.lru_cache(maxsize=None)
def _sc_deg(n_pad: int, nchunk: int):
    rpt = n_pad // NS
    mesh = plsc.VectorSubcoreMesh(core_axis_name="c", subcore_axis_name="s")

    def body(dst_hbm, ones_hbm, zeros_hbm, out_hbm, dst_v, ones_v, acc):
        cid = lax.axis_index("c")
        sid = lax.axis_index("s")
        pltpu.sync_copy(dst_hbm.at[sid], dst_v)
        pltpu.sync_copy(ones_hbm, ones_v)
        pltpu.sync_copy(zeros_hbm, acc.at[pl.ds(sid * rpt, rpt)])
        plsc.subcore_barrier()

        @pl.loop(cid, nchunk, step=NC)
        def _(j):
            pltpu.sync_copy(ones_v, acc.at[dst_v.at[j]], add=True)

        plsc.subcore_barrier()
        pltpu.sync_copy(acc.at[pl.ds(sid * rpt, rpt)],
                        out_hbm.at[cid, pl.ds(sid * rpt, rpt)])

    return pl.kernel(
        body,
        out_type=jax.ShapeDtypeStruct((NC, n_pad, DW), jnp.float32),
        mesh=mesh,
        compiler_params=pltpu.CompilerParams(use_tc_tiling_on_sc=False),
        scratch_types=[
            pltpu.VMEM((nchunk, K), jnp.int32),
            pltpu.VMEM((K, DW), jnp.float32),
            pltpu.VMEM_SHARED((n_pad, DW), jnp.float32),
        ],
    )


@functools.lru_cache(maxsize=None)
def _sc_scatter(n_pad: int, nchunk: int):
    rpt = n_pad // NS
    mesh = plsc.VectorSubcoreMesh(core_axis_name="c", subcore_axis_name="s")

    NB = 2
    assert nchunk % NB == 0 and nchunk > 2 * NB

    def body(s_hbm, src_hbm, dst_hbm, zeros_hbm, out_hbm,
             src_v, dst_v, bufs, gsems, acc):
        cid = lax.axis_index("c")
        sid = lax.axis_index("s")
        table = s_hbm.at[cid]
        pltpu.sync_copy(src_hbm.at[sid], src_v)
        pltpu.sync_copy(dst_hbm.at[sid], dst_v)
        pltpu.sync_copy(zeros_hbm, acc.at[pl.ds(sid * rpt, rpt)])
        plsc.subcore_barrier()

        def gather(j, b):
            pltpu.async_copy(table.at[src_v.at[j]], bufs[b], gsems[b])

        def gather_wait(j, b):
            pltpu.make_async_copy(
                table.at[src_v.at[j]], bufs[b], gsems[b]).wait()

        def scatter_sync(j, b):
            pltpu.sync_copy(bufs[b], acc.at[dst_v.at[j]], add=True)

        for b in range(NB):
            gather(b, b)

        @pl.loop(0, nchunk - NB, step=NB)
        def _(j0):
            for b in range(NB):
                j = j0 + b
                gather_wait(j, b)
                scatter_sync(j, b)
                gather(j + NB, b)

        for t in range(nchunk - NB, nchunk):
            gather_wait(t, t % NB)
            scatter_sync(t, t % NB)

        plsc.subcore_barrier()
        pltpu.sync_copy(acc.at[pl.ds(sid * rpt, rpt)],
                        out_hbm.at[cid, pl.ds(sid * rpt, rpt)])

    return pl.kernel(
        body,
        out_type=jax.ShapeDtypeStruct((NC, n_pad, DH), jnp.float32),
        mesh=mesh,
        compiler_params=pltpu.CompilerParams(use_tc_tiling_on_sc=False),
        scratch_types=[
            pltpu.VMEM((nchunk, K), jnp.int32),
            pltpu.VMEM((nchunk, K), jnp.int32),
            tuple(pltpu.VMEM((K, DH), jnp.float32) for _ in range(NB)),
            tuple(pltpu.SemaphoreType.DMA for _ in range(NB)),
            pltpu.VMEM_SHARED((n_pad, DH), jnp.float32),
        ],
    )


_RB = 2000


def _split_spec(n_pad, w):
    return pl.BlockSpec((NC, _RB, w), lambda i: (0, i, 0))


def _row_spec(w):
    return pl.BlockSpec((_RB, w), lambda i: (i, 0))


def _tc_pre(x, w1, deg):
    n, d = x.shape

    def body(x_ref, w_ref, deg_ref, s_ref, dinv_ref):
        dsum = deg_ref[0] + deg_ref[1] + 1.0
        dv = lax.rsqrt(dsum)
        dinv_ref[...] = dv
        res = jnp.dot(
            x_ref[...], w_ref[...], preferred_element_type=jnp.float32
        ) * dv[:, :1]
        s_ref[0] = res[:, :DH]
        s_ref[1] = res[:, DH:]

    return pl.pallas_call(
        body,
        grid=(n // _RB,),
        in_specs=[
            _row_spec(d),
            pl.BlockSpec((d, d), lambda i: (0, 0)),
            _split_spec(None, DW),
        ],
        out_specs=[
            _split_spec(None, DH),
            _row_spec(DW),
        ],
        out_shape=[
            jax.ShapeDtypeStruct((NC, n, DH), jnp.float32),
            jax.ShapeDtypeStruct((n, DW), jnp.float32),
        ],
    )(x, w1, deg)


def _tc_mid(p, s, dinv, b, h_prev, w_next):
    n, d = h_prev.shape

    def body(p_ref, s_ref, dinv_ref, b_ref, hp_ref, w_ref, h_ref, sn_ref):
        dv = dinv_ref[...][:, :1]
        agg = jnp.concatenate(
            [p_ref[0] + s_ref[0], p_ref[1] + s_ref[1]], axis=1
        ) * dv + b_ref[...]
        h = jnp.maximum(agg, 0.0) + hp_ref[...]
        h_ref[...] = h
        res = jnp.dot(
            h, w_ref[...], preferred_element_type=jnp.float32) * dv
        sn_ref[0] = res[:, :DH]
        sn_ref[1] = res[:, DH:]

    return pl.pallas_call(
        body,
        grid=(n // _RB,),
        in_specs=[
            _split_spec(None, DH),
            _split_spec(None, DH),
            _row_spec(DW),
            pl.BlockSpec((1, d), lambda i: (0, 0)),
            _row_spec(d),
            pl.BlockSpec((d, d), lambda i: (0, 0)),
        ],
        out_specs=[
            _row_spec(d),
            _split_spec(None, DH),
        ],
        out_shape=[
            jax.ShapeDtypeStruct((n, d), jnp.float32),
            jax.ShapeDtypeStruct((NC, n, DH), jnp.float32),
        ],
    )(p, s, dinv, b, h_prev, w_next)


def _tc_post(p, s, dinv, b, n, d):
    def body(p_ref, s_ref, dinv_ref, b_ref, o_ref):
        dv = dinv_ref[...][:, :1]
        o_ref[...] = jnp.concatenate(
            [p_ref[0] + s_ref[0], p_ref[1] + s_ref[1]], axis=1
        ) * dv + b_ref[...]

    return pl.pallas_call(
        body,
        grid=(n // _RB,),
        in_specs=[
            _split_spec(None, DH),
            _split_spec(None, DH),
            _row_spec(DW),
            pl.BlockSpec((1, d), lambda i: (0, 0)),
        ],
        out_specs=_row_spec(d),
        out_shape=jax.ShapeDtypeStruct((n, d), jnp.float32),
    )(p, s, dinv, b)


def kernel(x, edge_index, W1, b1, W2, b2, W3, b3):
    n, d = x.shape
    e = edge_index.shape[1]

    n_pad = -(-(n + 1) // (NS * 8)) * (NS * 8)
    rpt = n_pad // NS

    nchunk = -(-(-(-e // (NS * K))) // 2) * 2
    e_pad = NS * nchunk * K

    src = jnp.concatenate(
        [edge_index[0], jnp.zeros((e_pad - e,), jnp.int32)]).reshape(
            NS, nchunk, K)
    dst = jnp.concatenate(
        [edge_index[1], jnp.full((e_pad - e,), n, jnp.int32)]).reshape(
            NS, nchunk, K)

    ones_w = jnp.ones((K, DW), jnp.float32)
    zeros_w = jnp.zeros((rpt, DW), jnp.float32)
    zeros_h = jnp.zeros((rpt, DH), jnp.float32)

    deg = _sc_deg(n_pad, nchunk)(dst, ones_w, zeros_w)
    s1, dinv = _tc_pre(x, W1, deg)

    scat = _sc_scatter(n_pad, nchunk)
    b1r, b2r, b3r = (b.reshape(1, d) for b in (b1, b2, b3))

    p = scat(s1, src, dst, zeros_h)
    h1, s2 = _tc_mid(p, s1, dinv, b1r, x, W2)
    p = scat(s2, src, dst, zeros_h)
    h2, s3 = _tc_mid(p, s2, dinv, b2r, h1, W3)
    p = scat(s3, src, dst, zeros_h)
    return _tc_post(p, s3, dinv, b3r, n, d)
